# baseline (speedup 1.0000x reference)
import functools

import jax
import jax.numpy as jnp
import numpy as np
from jax.experimental import pallas as pl
from jax.experimental.pallas import tpu as pltpu

N = 500000
GRAD_THRESHOLD = 0.5
MIN_OPACITY = 0.05
LOG2 = float(np.log(2.0))

LANES = 128
ROWS = (N + LANES - 1) // LANES
NPAD = ROWS * LANES
BBLK = 2000


def _median_kernel(st_ref, thr_ref):
    e0 = jnp.exp(st_ref[0])
    e1 = jnp.exp(st_ref[1])
    e2 = jnp.exp(st_ref[2])
    sn2 = e0 * e0 + e1 * e1 + e2 * e2
    bits = jax.lax.bitcast_convert_type(sn2, jnp.int32)

    def find(k):
        def body(i, prefix):
            b = 30 - i
            cand = prefix | (1 << b)
            cnt = jnp.sum((bits < cand).astype(jnp.int32))
            return jnp.where(cnt <= k, cand, prefix)

        return jax.lax.fori_loop(0, 31, body, 0)

    t0 = find(N // 2 - 1)
    t1 = find(N // 2)
    v0 = jax.lax.bitcast_convert_type(t0, jnp.float32)
    v1 = jax.lax.bitcast_convert_type(t1, jnp.float32)
    thr_ref[...] = jnp.full((1, 1), 0.5 * (jnp.sqrt(v0) + jnp.sqrt(v1)),
                            jnp.float32)


def _main_kernel(thr_ref, pos_ref, sc_ref, rot_ref, op_ref, dc_ref,
                 rest_ref, ga_ref, gc_ref, noise_ref, out_ref):
    thr = thr_ref[0, 0]
    counts = jnp.maximum(gc_ref[...].astype(jnp.float32), 1.0)
    avg = ga_ref[...] / counts
    gn = jnp.sqrt(jnp.sum(avg * avg, axis=1, keepdims=True))
    large = gn >= GRAD_THRESHOLD

    sc = sc_ref[...]
    asc = jnp.exp(sc)
    sn = jnp.sqrt(jnp.sum(asc * asc, axis=1, keepdims=True))
    clone = large & (sn <= thr)
    split = large & (sn > thr)
    act_op = jax.nn.sigmoid(op_ref[...])
    keep = jnp.logical_not((act_op < MIN_OPACITY) | split)

    pos = pos_ref[...]
    rot = rot_ref[...]
    op = op_ref[...]
    dc = dc_ref[...]
    rest = rest_ref[...]
    P = jnp.concatenate([pos, sc, rot, op, dc, rest], axis=1)
    out_ref[0] = jnp.where(keep, P, 0.0)
    out_ref[1] = jnp.where(clone, P, 0.0)
    sp_sc = sc - LOG2
    for i in range(2):
        sp_pos = pos + noise_ref[i] * asc
        Pi = jnp.concatenate([sp_pos, sp_sc, rot, op, dc, rest], axis=1)
        out_ref[2 + i] = jnp.where(split, Pi, 0.0)


def kernel(positions, scales, rotations, opacities, sh_dc, sh_rest,
           grad_accum, grad_count, split_noise):
    f32 = jnp.float32
    st = jnp.transpose(scales)
    st = jnp.pad(st, ((0, 0), (0, NPAD - N)), constant_values=100.0)
    st = st.reshape(3, ROWS, LANES)
    thr = pl.pallas_call(
        _median_kernel,
        out_shape=jax.ShapeDtypeStruct((1, 1), f32),
    )(st)

    gc2 = grad_count.reshape(N, 1)
    B = BBLK
    grid = (N // B,)

    def row_spec(w):
        return pl.BlockSpec((B, w), lambda i: (i, 0))

    out4 = pl.pallas_call(
        _main_kernel,
        grid=grid,
        in_specs=[
            pl.BlockSpec((1, 1), lambda i: (0, 0)),
            row_spec(3),
            row_spec(3),
            row_spec(4),
            row_spec(1),
            row_spec(3),
            row_spec(9),
            row_spec(2),
            row_spec(1),
            pl.BlockSpec((2, B, 3), lambda i: (0, i, 0)),
        ],
        out_specs=pl.BlockSpec((4, B, 23), lambda i: (0, i, 0)),
        out_shape=jax.ShapeDtypeStruct((4, N, 23), f32),
    )(thr, positions, scales, rotations, opacities, sh_dc, sh_rest,
      grad_accum, gc2, split_noise)
    return out4.reshape(4 * N, 23)
